# baseline (speedup 1.0000x reference)
import functools

import jax
import jax.numpy as jnp
from jax import lax
from jax.experimental import pallas as pl
from jax.experimental.pallas import tpu as pltpu
from jax.experimental.pallas import tpu_sc as plsc

B = 16384
DIM = 32
WEIGHT_DECAY = 0.0001
NC = 2
NS = 16
NW = NC * NS
BPW = B // NW
NCHUNK = BPW // 128
NG = BPW // 16


def _sc_body(u2, i2, j2, r1, U, V, biasV, r_out, reg_out,
             idx_u, idx_i, idx_j, rows_u, rows_i, rows_j,
             bias_i, bias_j, r_v, out_r, reg_s, sem):
    wid = lax.axis_index("s") * NC + lax.axis_index("c")
    base = wid * BPW

    pltpu.sync_copy(u2.at[pl.ds(wid * NCHUNK, NCHUNK)], idx_u)
    pltpu.sync_copy(i2.at[pl.ds(wid * NCHUNK, NCHUNK)], idx_i)
    pltpu.sync_copy(j2.at[pl.ds(wid * NCHUNK, NCHUNK)], idx_j)

    descs = []
    for k in range(NCHUNK):
        sl = pl.ds(k * 128, 128)
        descs.append(pltpu.async_copy(U.at[idx_u.at[k]], rows_u.at[sl], sem))
        descs.append(pltpu.async_copy(V.at[idx_i.at[k]], rows_i.at[sl], sem))
        descs.append(pltpu.async_copy(V.at[idx_j.at[k]], rows_j.at[sl], sem))
        descs.append(pltpu.async_copy(biasV.at[idx_i.at[k]], bias_i.at[sl], sem))
        descs.append(pltpu.async_copy(biasV.at[idx_j.at[k]], bias_j.at[sl], sem))
    pltpu.sync_copy(r1.at[pl.ds(base, BPW)], r_v)
    for d in descs:
        d.wait()

    lane = lax.iota(jnp.int32, 16)
    zero = jnp.zeros((16,), jnp.float32)

    def g_body(g, sq):
        row = g * 16 + lane
        acc_ui = zero
        acc_uj = zero
        for d in range(DIM):
            col = jnp.full((16,), d, jnp.int32)
            ue = plsc.load_gather(rows_u, [row, col])
            ie = plsc.load_gather(rows_i, [row, col])
            je = plsc.load_gather(rows_j, [row, col])
            acc_ui = acc_ui + ue * ie
            acc_uj = acc_uj + ue * je
            sq = sq + ue * ue + ie * ie + je * je
        gsl = pl.ds(g * 16, 16)
        bi = bias_i[gsl]
        bj = bias_j[gsl]
        r = r_v[gsl]
        pw = (jnp.int32(1) << r).astype(jnp.float32)
        barr = jnp.where(r == 5, jnp.float32(1.0),
                         (pw - 1.0) * jnp.float32(1.0 / 32.0))
        out_r[gsl] = (acc_ui - acc_uj + bi - bj) * barr
        return sq + bi * bi + bj * bj

    sq = lax.fori_loop(0, NG, g_body, zero)
    reg_s[...] = sq
    pltpu.sync_copy(out_r, r_out.at[pl.ds(base, BPW)])
    pltpu.sync_copy(reg_s, reg_out.at[pl.ds(wid * 16, 16)])


@jax.jit
def _sc_gather(u2, i2, j2, r1, U, V, biasV):
    mesh = plsc.VectorSubcoreMesh(core_axis_name="c", subcore_axis_name="s",
                                  num_cores=NC, num_subcores=NS)
    f = pl.kernel(
        _sc_body,
        out_type=(jax.ShapeDtypeStruct((B,), jnp.float32),
                  jax.ShapeDtypeStruct((NW * 16,), jnp.float32)),
        mesh=mesh,
        compiler_params=pltpu.CompilerParams(needs_layout_passes=False,
                                             use_tc_tiling_on_sc=False),
        scratch_types=[
            pltpu.VMEM((NCHUNK, 128), jnp.int32),
            pltpu.VMEM((NCHUNK, 128), jnp.int32),
            pltpu.VMEM((NCHUNK, 128), jnp.int32),
            pltpu.VMEM((BPW, DIM), jnp.float32),
            pltpu.VMEM((BPW, DIM), jnp.float32),
            pltpu.VMEM((BPW, DIM), jnp.float32),
            pltpu.VMEM((BPW,), jnp.float32),
            pltpu.VMEM((BPW,), jnp.float32),
            pltpu.VMEM((BPW,), jnp.int32),
            pltpu.VMEM((BPW,), jnp.float32),
            pltpu.VMEM((16,), jnp.float32),
            pltpu.SemaphoreType.DMA,
        ],
    )
    return f(u2, i2, j2, r1, U, V, biasV)


def _tc_body(r_ref, reg_ref, o_ref):
    x = r_ref[...]
    log_sig = jnp.log(jax.nn.sigmoid(x))
    o_ref[0, 0] = (jnp.float32(WEIGHT_DECAY) * jnp.sum(reg_ref[...])
                   - jnp.sum(log_sig))


@jax.jit
def _tc_reduce(r_uij, reg):
    out = pl.pallas_call(
        _tc_body,
        out_shape=jax.ShapeDtypeStruct((1, 1), jnp.float32),
        out_specs=pl.BlockSpec(memory_space=pltpu.SMEM),
    )(r_uij.reshape(128, 128), reg.reshape(4, 128))
    return out[0, 0]


def kernel(u, i, r_ui, j, U, V, biasV):
    u = u.astype(jnp.int32)
    i = i.astype(jnp.int32)
    j = j.astype(jnp.int32)
    r_ui = r_ui.astype(jnp.int32)
    r_uij, reg = _sc_gather(u.reshape(-1, 128), i.reshape(-1, 128),
                            j.reshape(-1, 128), r_ui, U, V, biasV)
    return _tc_reduce(r_uij, reg)


# speedup vs baseline: 2.1617x; 2.1617x over previous
import functools

import jax
import jax.numpy as jnp
from jax import lax
from jax.experimental import pallas as pl
from jax.experimental.pallas import tpu as pltpu
from jax.experimental.pallas import tpu_sc as plsc

B = 16384
DIM = 32
WEIGHT_DECAY = 0.0001
NC = 2
NS = 16
NW = NC * NS
BPW = B // NW
NCHUNK = BPW // 128
NG = BPW // 16
CHUNK = 4
NBLK = 3 * CHUNK


def _sc_body(u1, i1, j1, r1, Ut, Vt, biasV, r_out, reg_out,
             idx_u, idx_i, idx_j, blks, rows_u, rows_i, rows_j,
             bias_i, bias_j, r_v, out_r, reg_s, sem, bsem):
    wid = lax.axis_index("s") * NC + lax.axis_index("c")
    base = wid * BPW

    pltpu.sync_copy(u1.at[pl.ds(base, BPW)], idx_u.at[pl.ds(0, BPW)])
    pltpu.sync_copy(i1.at[pl.ds(base, BPW)], idx_i.at[pl.ds(0, BPW)])
    pltpu.sync_copy(j1.at[pl.ds(base, BPW)], idx_j.at[pl.ds(0, BPW)])
    pltpu.sync_copy(r1.at[pl.ds(base, BPW)], r_v)

    bias_descs = []
    for k in range(NCHUNK):
        sl = pl.ds(k * 128, 128)
        bias_descs.append(
            pltpu.async_copy(biasV.at[idx_i.at[sl]], bias_i.at[sl], bsem))
        bias_descs.append(
            pltpu.async_copy(biasV.at[idx_j.at[sl]], bias_j.at[sl], bsem))

    lane = lax.iota(jnp.int32, 16)
    zero = jnp.zeros((16,), jnp.float32)

    def fetch_body(g2, carry):
        descs = []
        lanes = []
        for t in range(CHUNK):
            e = g2 * CHUNK + t
            ru = idx_u[pl.ds(e, 16)][0]
            ri = idx_i[pl.ds(e, 16)][0]
            rj = idx_j[pl.ds(e, 16)][0]
            bu = pl.multiple_of((ru // 128) * 128, 128)
            bi = pl.multiple_of((ri // 128) * 128, 128)
            bj = pl.multiple_of((rj // 128) * 128, 128)
            descs.append(pltpu.async_copy(
                Ut.at[:, pl.ds(bu, 128)], blks.at[3 * t + 0], sem))
            descs.append(pltpu.async_copy(
                Vt.at[:, pl.ds(bi, 128)], blks.at[3 * t + 1], sem))
            descs.append(pltpu.async_copy(
                Vt.at[:, pl.ds(bj, 128)], blks.at[3 * t + 2], sem))
            lanes.append((ru - bu, ri - bi, rj - bj))
        for d in descs:
            d.wait()
        for t in range(CHUNK):
            e = g2 * CHUNK + t
            lu, li, lj = lanes[t]
            for tb, (l, rows) in enumerate(
                    ((lu, rows_u), (li, rows_i), (lj, rows_j))):
                slot = jnp.full((16,), 3 * t + tb, jnp.int32)
                lv = jnp.full((16,), l, jnp.int32)
                lo = plsc.load_gather(blks, [slot, lane, lv])
                hi = plsc.load_gather(blks, [slot, lane + 16, lv])
                rows[pl.ds(e * DIM, 16)] = lo
                rows[pl.ds(e * DIM + 16, 16)] = hi
        return carry

    lax.fori_loop(0, BPW // CHUNK, fetch_body, 0)
    for d in bias_descs:
        d.wait()

    def g_body(g, sq):
        acc_ui = zero
        acc_uj = zero
        gbase = g * 16 * DIM
        for d in range(DIM):
            idx = lane * DIM + (gbase + d)
            ue = plsc.load_gather(rows_u, [idx])
            ie = plsc.load_gather(rows_i, [idx])
            je = plsc.load_gather(rows_j, [idx])
            acc_ui = acc_ui + ue * ie
            acc_uj = acc_uj + ue * je
            sq = sq + ue * ue + ie * ie + je * je
        gsl = pl.ds(g * 16, 16)
        bi = bias_i[gsl]
        bj = bias_j[gsl]
        r = r_v[gsl]
        pw = (jnp.int32(1) << r).astype(jnp.float32)
        barr = jnp.where(r == 5, jnp.float32(1.0),
                         (pw - 1.0) * jnp.float32(1.0 / 32.0))
        out_r[gsl] = (acc_ui - acc_uj + bi - bj) * barr
        return sq + bi * bi + bj * bj

    sq = lax.fori_loop(0, NG, g_body, zero)
    reg_s[...] = sq
    pltpu.sync_copy(out_r, r_out.at[pl.ds(base, BPW)])
    pltpu.sync_copy(reg_s, reg_out.at[pl.ds(wid * 16, 16)])


@jax.jit
def _sc_gather(u1, i1, j1, r1, Ut, Vt, biasV):
    mesh = plsc.VectorSubcoreMesh(core_axis_name="c", subcore_axis_name="s",
                                  num_cores=NC, num_subcores=NS)
    f = pl.kernel(
        _sc_body,
        out_type=(jax.ShapeDtypeStruct((B,), jnp.float32),
                  jax.ShapeDtypeStruct((NW * 16,), jnp.float32)),
        mesh=mesh,
        compiler_params=pltpu.CompilerParams(needs_layout_passes=False,
                                             use_tc_tiling_on_sc=True),
        scratch_types=[
            pltpu.VMEM((BPW + 16,), jnp.int32),
            pltpu.VMEM((BPW + 16,), jnp.int32),
            pltpu.VMEM((BPW + 16,), jnp.int32),
            pltpu.VMEM((NBLK, DIM, 128), jnp.float32),
            pltpu.VMEM((BPW * DIM,), jnp.float32),
            pltpu.VMEM((BPW * DIM,), jnp.float32),
            pltpu.VMEM((BPW * DIM,), jnp.float32),
            pltpu.VMEM((BPW,), jnp.float32),
            pltpu.VMEM((BPW,), jnp.float32),
            pltpu.VMEM((BPW,), jnp.int32),
            pltpu.VMEM((BPW,), jnp.float32),
            pltpu.VMEM((16,), jnp.float32),
            pltpu.SemaphoreType.DMA,
            pltpu.SemaphoreType.DMA,
        ],
    )
    return f(u1, i1, j1, r1, Ut, Vt, biasV)


def _tc_body(r_ref, reg_ref, o_ref):
    x = r_ref[...]
    log_sig = jnp.log(jax.nn.sigmoid(x))
    o_ref[0, 0] = (jnp.float32(WEIGHT_DECAY) * jnp.sum(reg_ref[...])
                   - jnp.sum(log_sig))


@jax.jit
def _tc_reduce(r_uij, reg):
    out = pl.pallas_call(
        _tc_body,
        out_shape=jax.ShapeDtypeStruct((1, 1), jnp.float32),
        out_specs=pl.BlockSpec(memory_space=pltpu.SMEM),
    )(r_uij.reshape(128, 128), reg.reshape(4, 128))
    return out[0, 0]


def kernel(u, i, r_ui, j, U, V, biasV):
    u = u.astype(jnp.int32)
    i = i.astype(jnp.int32)
    j = j.astype(jnp.int32)
    r_ui = r_ui.astype(jnp.int32)
    r_uij, reg = _sc_gather(u, i, j, r_ui, U.T, V.T, biasV)
    return _tc_reduce(r_uij, reg)


# speedup vs baseline: 2.4077x; 1.1138x over previous
import functools

import jax
import jax.numpy as jnp
from jax import lax
from jax.experimental import pallas as pl
from jax.experimental.pallas import tpu as pltpu
from jax.experimental.pallas import tpu_sc as plsc

B = 16384
DIM = 32
WEIGHT_DECAY = 0.0001
NC = 2
NS = 16
NW = NC * NS
BPW = B // NW
NCHUNK = BPW // 128
NG = BPW // 16
HE = 2
NBLK = 2 * 3 * HE


def _sc_body(u1, i1, j1, r1, Ut, Vt, biasV, r_out, reg_out,
             idx_u, idx_i, idx_j, blks, rows_u, rows_i, rows_j,
             bias_i, bias_j, r_v, out_r, reg_s, semA, semB, bsem):
    wid = lax.axis_index("s") * NC + lax.axis_index("c")
    base = wid * BPW

    pltpu.sync_copy(u1.at[pl.ds(base, BPW)], idx_u.at[pl.ds(0, BPW)])
    pltpu.sync_copy(i1.at[pl.ds(base, BPW)], idx_i.at[pl.ds(0, BPW)])
    pltpu.sync_copy(j1.at[pl.ds(base, BPW)], idx_j.at[pl.ds(0, BPW)])
    pltpu.sync_copy(r1.at[pl.ds(base, BPW)], r_v)

    bias_descs = []
    for k in range(NCHUNK):
        sl = pl.ds(k * 128, 128)
        bias_descs.append(
            pltpu.async_copy(biasV.at[idx_i.at[sl]], bias_i.at[sl], bsem))
        bias_descs.append(
            pltpu.async_copy(biasV.at[idx_j.at[sl]], bias_j.at[sl], bsem))

    lane = lax.iota(jnp.int32, 16)
    zero = jnp.zeros((16,), jnp.float32)
    reg_s[...] = zero

    def read_idx(e):
        ru = idx_u[pl.ds(e, 16)][0]
        ri = idx_i[pl.ds(e, 16)][0]
        rj = idx_j[pl.ds(e, 16)][0]
        return ru, ri, rj

    def fire(eb, half, sem):
        for t in range(HE):
            ru, ri, rj = read_idx(eb + t)
            bu = pl.multiple_of((ru // 128) * 128, 128)
            bi = pl.multiple_of((ri // 128) * 128, 128)
            bj = pl.multiple_of((rj // 128) * 128, 128)
            s0 = half * 3 * HE + t * 3
            pltpu.async_copy(Ut.at[:, pl.ds(bu, 128)], blks.at[s0 + 0], sem)
            pltpu.async_copy(Vt.at[:, pl.ds(bi, 128)], blks.at[s0 + 1], sem)
            pltpu.async_copy(Vt.at[:, pl.ds(bj, 128)], blks.at[s0 + 2], sem)

    def drain(half, sem):
        for s in range(3 * HE):
            pltpu.make_async_copy(
                Ut.at[:, pl.ds(0, 128)],
                blks.at[half * 3 * HE + s], sem).wait()

    def extract(eb, half):
        for t in range(HE):
            e = eb + t
            ru, ri, rj = read_idx(e)
            lu = lax.rem(ru, jnp.int32(128))
            li = lax.rem(ri, jnp.int32(128))
            lj = lax.rem(rj, jnp.int32(128))
            s0 = half * 3 * HE + t * 3
            for tb, (l, rows) in enumerate(
                    ((lu, rows_u), (li, rows_i), (lj, rows_j))):
                slot = jnp.full((16,), s0 + tb, jnp.int32)
                lv = jnp.full((16,), l, jnp.int32)
                lo = plsc.load_gather(blks, [slot, lane, lv])
                hi = plsc.load_gather(blks, [slot, lane + 16, lv])
                rows[pl.ds(e * DIM, 16)] = lo
                rows[pl.ds(e * DIM + 16, 16)] = hi

    def dot_group(g):
        acc_ui = zero
        acc_uj = zero
        sq = zero
        gbase = g * (16 * DIM)
        for d in range(DIM):
            idx = lane * DIM + (gbase + d)
            ue = plsc.load_gather(rows_u, [idx])
            ie = plsc.load_gather(rows_i, [idx])
            je = plsc.load_gather(rows_j, [idx])
            acc_ui = acc_ui + ue * ie
            acc_uj = acc_uj + ue * je
            sq = sq + ue * ue + ie * ie + je * je
        gsl = pl.ds(g * 16, 16)
        bi = bias_i[gsl]
        bj = bias_j[gsl]
        r = r_v[gsl]
        pw = (jnp.int32(1) << r).astype(jnp.float32)
        barr = jnp.where(r == 5, jnp.float32(1.0),
                         (pw - 1.0) * jnp.float32(1.0 / 32.0))
        out_r[gsl] = (acc_ui - acc_uj + bi - bj) * barr
        reg_s[...] = reg_s[...] + sq + bi * bi + bj * bj

    fire(0, 0, semA)
    fire(HE, 1, semB)
    for d in bias_descs:
        d.wait()

    W = 2 * HE

    def body(k, carry):
        eb = k * W
        drain(0, semA)
        extract((k - 1) * W, 0)
        fire(eb, 0, semA)
        drain(1, semB)
        extract((k - 1) * W + HE, 1)
        fire(eb + HE, 1, semB)

        @pl.when((lax.rem(k, jnp.int32(4)) == 0) & (k >= 4))
        def _():
            dot_group(k // 4 - 1)
        return carry

    lax.fori_loop(1, BPW // W, body, 0)
    drain(0, semA)
    extract(BPW - W, 0)
    drain(1, semB)
    extract(BPW - HE, 1)
    dot_group(NG - 1)

    pltpu.sync_copy(out_r, r_out.at[pl.ds(base, BPW)])
    pltpu.sync_copy(reg_s, reg_out.at[pl.ds(wid * 16, 16)])


@jax.jit
def _sc_gather(u1, i1, j1, r1, Ut, Vt, biasV):
    mesh = plsc.VectorSubcoreMesh(core_axis_name="c", subcore_axis_name="s",
                                  num_cores=NC, num_subcores=NS)
    f = pl.kernel(
        _sc_body,
        out_type=(jax.ShapeDtypeStruct((B,), jnp.float32),
                  jax.ShapeDtypeStruct((NW * 16,), jnp.float32)),
        mesh=mesh,
        compiler_params=pltpu.CompilerParams(needs_layout_passes=False,
                                             use_tc_tiling_on_sc=True),
        scratch_types=[
            pltpu.VMEM((BPW + 16,), jnp.int32),
            pltpu.VMEM((BPW + 16,), jnp.int32),
            pltpu.VMEM((BPW + 16,), jnp.int32),
            pltpu.VMEM((NBLK, DIM, 128), jnp.float32),
            pltpu.VMEM((BPW * DIM,), jnp.float32),
            pltpu.VMEM((BPW * DIM,), jnp.float32),
            pltpu.VMEM((BPW * DIM,), jnp.float32),
            pltpu.VMEM((BPW,), jnp.float32),
            pltpu.VMEM((BPW,), jnp.float32),
            pltpu.VMEM((BPW,), jnp.int32),
            pltpu.VMEM((BPW,), jnp.float32),
            pltpu.VMEM((16,), jnp.float32),
            pltpu.SemaphoreType.DMA,
            pltpu.SemaphoreType.DMA,
            pltpu.SemaphoreType.DMA,
        ],
    )
    return f(u1, i1, j1, r1, Ut, Vt, biasV)


def _tc_body(r_ref, reg_ref, o_ref):
    x = r_ref[...]
    log_sig = jnp.log(jax.nn.sigmoid(x))
    o_ref[0, 0] = (jnp.float32(WEIGHT_DECAY) * jnp.sum(reg_ref[...])
                   - jnp.sum(log_sig))


@jax.jit
def _tc_reduce(r_uij, reg):
    out = pl.pallas_call(
        _tc_body,
        out_shape=jax.ShapeDtypeStruct((1, 1), jnp.float32),
        out_specs=pl.BlockSpec(memory_space=pltpu.SMEM),
    )(r_uij.reshape(128, 128), reg.reshape(4, 128))
    return out[0, 0]


def kernel(u, i, r_ui, j, U, V, biasV):
    u = u.astype(jnp.int32)
    i = i.astype(jnp.int32)
    j = j.astype(jnp.int32)
    r_ui = r_ui.astype(jnp.int32)
    r_uij, reg = _sc_gather(u, i, j, r_ui, U.T, V.T, biasV)
    return _tc_reduce(r_uij, reg)
